# baseline (speedup 1.0000x reference)
import functools

import jax
import jax.numpy as jnp
from jax import lax
from jax.experimental import pallas as pl
from jax.experimental.pallas import tpu as pltpu
from jax.experimental.pallas import tpu_sc as plsc

MASK_LO = 900000
DIM = 64
CHUNK = 320
NBUF = 2


def _sc_embed(idx, W_main, W_mask):
    N = idx.shape[0]
    info = plsc.get_sparse_core_info()
    NC, NS, L = info.num_cores, info.num_subcores, info.num_lanes
    NW = NC * NS
    assert N % (NW * CHUNK * NBUF) == 0
    per_w = N // NW
    n_chunks = per_w // CHUNK
    n_vecs = CHUNK // L

    mesh = plsc.VectorSubcoreMesh(core_axis_name="c", subcore_axis_name="s")

    scratch = [pltpu.VMEM((per_w,), jnp.int32)]
    scratch += [pltpu.VMEM((CHUNK,), jnp.int32) for _ in range(NBUF)]
    scratch += [pltpu.VMEM((CHUNK,), jnp.int32) for _ in range(NBUF)]
    scratch += [pltpu.VMEM((CHUNK,), jnp.int32) for _ in range(NBUF)]
    scratch += [pltpu.VMEM((CHUNK,), jnp.int32) for _ in range(NBUF)]
    scratch += [pltpu.VMEM((CHUNK, DIM), jnp.float32) for _ in range(NBUF)]
    scratch += [pltpu.VMEM((CHUNK, DIM), jnp.float32) for _ in range(NBUF)]
    scratch += [pltpu.SMEM((2 * NBUF,), jnp.int32)]
    scratch += [pltpu.SemaphoreType.DMA] * (4 * NBUF + 1)

    @functools.partial(
        pl.kernel,
        out_type=jax.ShapeDtypeStruct((N, DIM), jnp.float32),
        mesh=mesh,
        scratch_types=scratch,
        compiler_params=pltpu.CompilerParams(use_tc_tiling_on_sc=False),
    )
    def k(idx_hbm, wmain_hbm, wmask_hbm, out_hbm, slab, *rest):
        idxm = rest[0:NBUF]
        midx = rest[NBUF:2 * NBUF]
        gposA = rest[2 * NBUF:3 * NBUF]
        gposB = rest[3 * NBUF:4 * NBUF]
        rows = rest[4 * NBUF:5 * NBUF]
        mrows = rest[5 * NBUF:6 * NBUF]
        fl = rest[6 * NBUF]
        s_gm = rest[6 * NBUF + 1:7 * NBUF + 1]
        s_gk = rest[7 * NBUF + 1:8 * NBUF + 1]
        s_sa = rest[8 * NBUF + 1:9 * NBUF + 1]
        s_sb = rest[9 * NBUF + 1:10 * NBUF + 1]
        s_slab = rest[10 * NBUF + 1]

        wid = lax.axis_index("s") * NC + lax.axis_index("c")
        wbase = wid * per_w
        lane = lax.iota(jnp.int32, L)
        rots = [((lane + sh) & (L - 1)).astype(jnp.int32) for sh in (8, 4, 2, 1)]

        pltpu.async_copy(idx_hbm.at[pl.ds(wbase, per_w)], slab, s_slab).wait()

        def wait_sa(j):
            pltpu.make_async_copy(rows[j], out_hbm.at[gposA[j]], s_sa[j]).wait()

        def wait_sb(j):
            pltpu.make_async_copy(mrows[j], out_hbm.at[gposB[j]], s_sb[j]).wait()

        def issue(c, j):
            cbase = wbase + c * CHUNK

            def scan_vec(v, carry):
                mm, mn = carry
                a = slab[pl.ds(c * CHUNK + v * L, L)]
                is_m = a >= MASK_LO
                pos = v * L + lane
                keym = jnp.where(is_m, (pos << 17) | (a - MASK_LO), -1)
                keyn = jnp.where(is_m, -1, (pos << 20) | a)
                return (jnp.maximum(mm, keym), jnp.maximum(mn, keyn))

            init = jnp.full((L,), -1, jnp.int32)
            Mm, Mn = lax.fori_loop(0, n_vecs, scan_vec, (init, init),
                                   unroll=2)
            for r in rots:
                Mm = jnp.maximum(Mm, Mm.at[r].get(mode="promise_in_bounds"))
                Mn = jnp.maximum(Mn, Mn.at[r].get(mode="promise_in_bounds"))
            fl[j] = (Mm[0] >= 0).astype(jnp.int32)
            fl[NBUF + j] = (Mn[0] >= 0).astype(jnp.int32)
            Mmc = jnp.maximum(Mm, 0)
            Mnc = jnp.maximum(Mn, 0)
            fm_pos, fm_midx = Mmc >> 17, Mmc & 0x1FFFF
            fn_pos, fn_id = Mnc >> 20, Mnc & 0xFFFFF

            def fix_vec(v, _):
                a = slab[pl.ds(c * CHUNK + v * L, L)]
                is_m = a >= MASK_LO
                pos = v * L + lane
                sl = pl.ds(v * L, L)
                idxm[j][sl] = jnp.where(is_m, fn_id, a)
                midx[j][sl] = jnp.where(is_m, a - MASK_LO, fm_midx)
                gposA[j][sl] = cbase + jnp.where(is_m, fn_pos, pos)
                gposB[j][sl] = cbase + jnp.where(is_m, pos, fm_pos)
                return 0

            lax.fori_loop(0, n_vecs, fix_vec, 0, unroll=2)
            pltpu.async_copy(wmain_hbm.at[idxm[j]], rows[j], s_gm[j])
            pltpu.async_copy(wmask_hbm.at[midx[j]], mrows[j], s_gk[j])

        def flush(c, j):
            pltpu.make_async_copy(
                wmain_hbm.at[idxm[j]], rows[j], s_gm[j]).wait()
            pltpu.make_async_copy(
                wmask_hbm.at[midx[j]], mrows[j], s_gk[j]).wait()

            @pl.when(fl[NBUF + j] != 0)
            def _():
                pltpu.async_copy(rows[j], out_hbm.at[gposA[j]], s_sa[j])

            @pl.when(fl[j] != 0)
            def _():
                pltpu.async_copy(mrows[j], out_hbm.at[gposB[j]], s_sb[j])

        def step(i, _):
            for j in range(NBUF):
                c = NBUF * i + j
                jw = (j - 1) % NBUF

                @pl.when(i >= 1)
                def _():
                    @pl.when(fl[NBUF + j] != 0)
                    def _():
                        wait_sa(j)

                    @pl.when(fl[j] != 0)
                    def _():
                        wait_sb(j)

                    issue(c, j)
                    flush(c - 1, jw)

                @pl.when(i == 0)
                def _():
                    issue(j, j)
                    if j >= 1:
                        flush(j - 1, j - 1)
            return 0

        lax.fori_loop(0, n_chunks // NBUF, step, 0)
        flush(n_chunks - 1, (n_chunks - 1) % NBUF)
        for cc in range(n_chunks - NBUF, n_chunks):
            @pl.when(fl[NBUF + cc % NBUF] != 0)
            def _(cc=cc):
                wait_sa(cc % NBUF)

            @pl.when(fl[cc % NBUF] != 0)
            def _(cc=cc):
                wait_sb(cc % NBUF)

    return k(idx, W_main, W_mask)


def kernel(input, W_main, W_mask):
    B, H = input.shape
    out = _sc_embed(input.reshape(B * H), W_main, W_mask)
    return out.reshape(B, H, DIM)


# speedup vs baseline: 1.7942x; 1.7942x over previous
import functools

import jax
import jax.numpy as jnp
from jax import lax
from jax.experimental import pallas as pl
from jax.experimental.pallas import tpu as pltpu
from jax.experimental.pallas import tpu_sc as plsc

MASK_LO = 900000
DIM = 64
CHUNK = 160
NBUF = 4
BLK = 32
NBLK = CHUNK // BLK


def _pc16(x):
    x = x - ((x >> 1) & 0x5555)
    x = (x & 0x3333) + ((x >> 2) & 0x3333)
    x = (x + (x >> 4)) & 0x0F0F
    return (x + (x >> 8)) & 0x1F


def _sc_embed(idx, W_main, W_mask):
    N = idx.shape[0]
    info = plsc.get_sparse_core_info()
    NC, NS, L = info.num_cores, info.num_subcores, info.num_lanes
    NW = NC * NS
    assert N % (NW * CHUNK * NBUF) == 0
    per_w = N // NW
    n_chunks = per_w // CHUNK
    n_vecs = CHUNK // L
    mpad = CHUNK + 2 * BLK

    mesh = plsc.VectorSubcoreMesh(core_axis_name="c", subcore_axis_name="s")

    scratch = [pltpu.VMEM((per_w,), jnp.int32)]
    scratch += [pltpu.VMEM((CHUNK,), jnp.int32) for _ in range(NBUF)]
    scratch += [pltpu.VMEM((mpad,), jnp.int32) for _ in range(NBUF)]
    scratch += [pltpu.VMEM((CHUNK,), jnp.int32) for _ in range(NBUF)]
    scratch += [pltpu.VMEM((mpad,), jnp.int32) for _ in range(NBUF)]
    scratch += [pltpu.VMEM((NBLK, BLK), jnp.int32) for _ in range(NBUF)]
    scratch += [pltpu.VMEM((CHUNK, DIM), jnp.float32) for _ in range(NBUF)]
    scratch += [pltpu.VMEM((CHUNK, DIM), jnp.float32) for _ in range(NBUF)]
    scratch += [pltpu.SMEM((2 * NBUF,), jnp.int32)]
    scratch += [pltpu.SemaphoreType.DMA] * (4 * NBUF + 1)

    @functools.partial(
        pl.kernel,
        out_type=jax.ShapeDtypeStruct((N, DIM), jnp.float32),
        mesh=mesh,
        scratch_types=scratch,
        compiler_params=pltpu.CompilerParams(use_tc_tiling_on_sc=False),
    )
    def k(idx_hbm, wmain_hbm, wmask_hbm, out_hbm, slab, *rest):
        idxm = rest[0:NBUF]
        midc = rest[NBUF:2 * NBUF]
        gposA = rest[2 * NBUF:3 * NBUF]
        gposB = rest[3 * NBUF:4 * NBUF]
        gposB2 = rest[4 * NBUF:5 * NBUF]
        rows = rest[5 * NBUF:6 * NBUF]
        mrows = rest[6 * NBUF:7 * NBUF]
        fl = rest[7 * NBUF]
        s_gm = rest[7 * NBUF + 1:8 * NBUF + 1]
        s_gk = rest[8 * NBUF + 1:9 * NBUF + 1]
        s_sa = rest[9 * NBUF + 1:10 * NBUF + 1]
        s_sb = rest[10 * NBUF + 1:11 * NBUF + 1]
        s_slab = rest[11 * NBUF + 1]

        wid = lax.axis_index("s") * NC + lax.axis_index("c")
        wbase = wid * per_w
        lane = lax.iota(jnp.int32, L)
        rots = [((lane + sh) & (L - 1)).astype(jnp.int32) for sh in (8, 4, 2, 1)]

        pltpu.async_copy(idx_hbm.at[pl.ds(wbase, per_w)], slab, s_slab).wait()

        def wait_sa(j):
            pltpu.make_async_copy(rows[j], out_hbm.at[gposA[j]], s_sa[j]).wait()

        def sb_desc(j, b):
            return pltpu.make_async_copy(
                mrows[j].at[pl.ds(b * BLK, BLK)],
                out_hbm.at[gposB2[j].at[b]], s_sb[j])

        def gk_desc(j, b):
            return pltpu.make_async_copy(
                wmask_hbm.at[midc[j].at[pl.ds(b * BLK, BLK)]],
                mrows[j].at[pl.ds(b * BLK, BLK)], s_gk[j])

        def issue(c, j):
            cbase = wbase + c * CHUNK

            def scan_vec(v, carry):
                mm, mn = carry
                a = slab[pl.ds(c * CHUNK + v * L, L)]
                is_m = a >= MASK_LO
                pos = v * L + lane
                keym = jnp.where(is_m, (pos << 17) | (a - MASK_LO), -1)
                keyn = jnp.where(is_m, -1, (pos << 20) | a)
                return (jnp.maximum(mm, keym), jnp.maximum(mn, keyn))

            init = jnp.full((L,), -1, jnp.int32)
            Mm, Mn = lax.fori_loop(0, n_vecs, scan_vec, (init, init),
                                   unroll=2)
            for r in rots:
                Mm = jnp.maximum(Mm, Mm.at[r].get(mode="promise_in_bounds"))
                Mn = jnp.maximum(Mn, Mn.at[r].get(mode="promise_in_bounds"))
            fl[NBUF + j] = (Mn[0] >= 0).astype(jnp.int32)
            Mmc = jnp.maximum(Mm, 0)
            Mnc = jnp.maximum(Mn, 0)
            fm_pos, fm_midx = Mmc >> 17, Mmc & 0x1FFFF
            fn_pos, fn_id = Mnc >> 20, Mnc & 0xFFFFF

            def fix_vec(v, cur):
                a = slab[pl.ds(c * CHUNK + v * L, L)]
                is_m = a >= MASK_LO
                pos = v * L + lane
                sl = pl.ds(v * L, L)
                idxm[j][sl] = jnp.where(is_m, fn_id, a)
                gposA[j][sl] = cbase + jnp.where(is_m, fn_pos, pos)
                bits = jnp.where(is_m, jnp.int32(1) << lane, jnp.int32(0))
                for r in rots:
                    bits = bits | bits.at[r].get(mode="promise_in_bounds")
                mb = bits[0]
                mi = a - MASK_LO
                gb = cbase + pos

                def tbody(_, carry):
                    cur2, bb = carry
                    low = bb & (-bb)
                    t = (lax.bitcast_convert_type(low.astype(jnp.float32),
                                                  jnp.int32) >> 23) - 127
                    tv = jnp.full((L,), t, jnp.int32)
                    midc[j][pl.ds(cur2, L)] = mi.at[tv].get(
                        mode="promise_in_bounds")
                    gposB[j][pl.ds(cur2, L)] = gb.at[tv].get(
                        mode="promise_in_bounds")
                    return (cur2 + 1, bb & (bb - 1))

                cur, _ = lax.fori_loop(0, _pc16(mb), tbody, (cur, mb))
                return cur

            nm = lax.fori_loop(0, n_vecs, fix_vec, jnp.int32(0))
            fl[j] = nm
            pltpu.async_copy(wmain_hbm.at[idxm[j]], rows[j], s_gm[j])
            for kk in range(BLK // L + 1):
                midc[j][pl.ds(nm + kk * L, L)] = fm_midx
                gposB[j][pl.ds(nm + kk * L, L)] = cbase + fm_pos
            for b in range(NBLK):
                for kk in range(BLK // L):
                    gposB2[j].at[b][pl.ds(kk * L, L)] = (
                        gposB[j][pl.ds(b * BLK + kk * L, L)])
            for b in range(NBLK):
                @pl.when(b * BLK < nm)
                def _(b=b):
                    gk_desc(j, b).start()

        def flush(c, j):
            nm = fl[j]
            pltpu.make_async_copy(
                wmain_hbm.at[idxm[j]], rows[j], s_gm[j]).wait()
            for b in range(NBLK):
                @pl.when(b * BLK < nm)
                def _(b=b):
                    gk_desc(j, b).wait()

            @pl.when(fl[NBUF + j] != 0)
            def _():
                pltpu.async_copy(rows[j], out_hbm.at[gposA[j]], s_sa[j])

            for b in range(NBLK):
                @pl.when(b * BLK < nm)
                def _(b=b):
                    sb_desc(j, b).start()

        def reuse_wait(j):
            @pl.when(fl[NBUF + j] != 0)
            def _():
                wait_sa(j)

            nm = fl[j]
            for b in range(NBLK):
                @pl.when(b * BLK < nm)
                def _(b=b):
                    sb_desc(j, b).wait()

        def step(i, _):
            for j in range(NBUF):
                c = NBUF * i + j
                jw = (j - 2) % NBUF

                @pl.when(i >= 1)
                def _():
                    reuse_wait(j)
                    issue(c, j)
                    flush(c - 2, jw)

                @pl.when(i == 0)
                def _():
                    issue(j, j)
                    if j >= 2:
                        flush(j - 2, j - 2)
            return 0

        lax.fori_loop(0, n_chunks // NBUF, step, 0)
        flush(n_chunks - 2, (n_chunks - 2) % NBUF)
        flush(n_chunks - 1, (n_chunks - 1) % NBUF)
        for cc in range(n_chunks - NBUF, n_chunks):
            reuse_wait(cc % NBUF)

    return k(idx, W_main, W_mask)


def kernel(input, W_main, W_mask):
    B, H = input.shape
    out = _sc_embed(input.reshape(B * H), W_main, W_mask)
    return out.reshape(B, H, DIM)
